# baseline (speedup 1.0000x reference)
import jax
import jax.numpy as jnp
from jax import lax
from jax.experimental import pallas as pl
from jax.experimental.pallas import tpu as pltpu

_EPS = 1e-12
_NPROTO = 16


def _guidance_kernel(p_ref, x_ref, o_ref):
    xb = x_ref[0]
    s = lax.dot_general(
        p_ref[...], xb,
        dimension_numbers=(((1,), (0,)), ((), ())),
        preferred_element_type=jnp.float32)
    ev_f = jnp.max(s[:_NPROTO], axis=0)
    ev_a = jnp.max(s[_NPROTO:], axis=0)
    norm2 = jnp.sum(xb * xb, axis=0)
    norm = jnp.maximum(jnp.sqrt(norm2), _EPS)
    o_ref[0, 0] = (ev_f - ev_a) / norm


def kernel(x, forgery_protos, authentic_protos):
    b, c, h, w = x.shape
    protos = jnp.concatenate([forgery_protos, authentic_protos], axis=0)

    out = pl.pallas_call(
        _guidance_kernel,
        grid=(b,),
        in_specs=[
            pl.BlockSpec((protos.shape[0], c), lambda i: (0, 0)),
            pl.BlockSpec((1, c, h, w), lambda i: (i, 0, 0, 0)),
        ],
        out_specs=pl.BlockSpec((1, 1, h, w), lambda i: (i, 0, 0, 0)),
        out_shape=jax.ShapeDtypeStruct((b, 1, h, w), jnp.float32),
        compiler_params=pltpu.CompilerParams(
            dimension_semantics=("parallel",),
        ),
    )(protos, x)

    return out


# speedup vs baseline: 1.2284x; 1.2284x over previous
import jax
import jax.numpy as jnp
from jax.experimental import pallas as pl
from jax.experimental.pallas import tpu as pltpu

_EPS = 1e-12
_NPROTO = 16


def _guidance_kernel(p_ref, x_ref, o_ref):
    nt = x_ref.shape[2]
    for t in range(nt):
        xt = x_ref[0, :, t, :]
        s = jnp.dot(p_ref[...], xt, preferred_element_type=jnp.float32)
        ev_f = jnp.max(s[:_NPROTO], axis=0)
        ev_a = jnp.max(s[_NPROTO:], axis=0)
        norm2 = jnp.sum(xt * xt, axis=0)
        norm = jnp.maximum(jnp.sqrt(norm2), _EPS)
        o_ref[0, 0, t] = (ev_f - ev_a) / norm


def kernel(x, forgery_protos, authentic_protos):
    b, c, h, w = x.shape
    protos = jnp.concatenate([forgery_protos, authentic_protos], axis=0)
    xv = x.reshape(b, c, h // 2, 2 * w)

    out = pl.pallas_call(
        _guidance_kernel,
        grid=(b,),
        in_specs=[
            pl.BlockSpec((protos.shape[0], c), lambda i: (0, 0)),
            pl.BlockSpec((1, c, h // 2, 2 * w), lambda i: (i, 0, 0, 0)),
        ],
        out_specs=pl.BlockSpec((1, 1, h // 2, 2 * w), lambda i: (i, 0, 0, 0)),
        out_shape=jax.ShapeDtypeStruct((b, 1, h // 2, 2 * w), jnp.float32),
        compiler_params=pltpu.CompilerParams(
            dimension_semantics=("parallel",),
        ),
    )(protos, xv)

    return out.reshape(b, 1, h, w)


# speedup vs baseline: 2.0203x; 1.6447x over previous
import jax
import jax.numpy as jnp
from jax.experimental import pallas as pl
from jax.experimental.pallas import tpu as pltpu

_EPS = 1e-12
_NPROTO = 16
_BBLK = 2


def _guidance_kernel(p_ref, x_ref, o_ref):
    for bi in range(_BBLK):
        xb = x_ref[bi]
        s = jnp.dot(p_ref[...], xb, preferred_element_type=jnp.float32)
        ev_f = jnp.max(s[:_NPROTO], axis=0)
        ev_a = jnp.max(s[_NPROTO:], axis=0)
        xf = xb.astype(jnp.float32)
        norm2 = jnp.sum(xf * xf, axis=0)
        norm = jnp.maximum(jnp.sqrt(norm2), _EPS)
        o_ref[bi] = ((ev_f - ev_a) / norm)[None, :]


def kernel(x, forgery_protos, authentic_protos):
    b, c, h, w = x.shape
    hw = h * w
    protos = jnp.concatenate([forgery_protos, authentic_protos], axis=0)
    x3 = x.reshape(b, c, hw).astype(jnp.bfloat16)
    protos = protos.astype(jnp.bfloat16)

    out = pl.pallas_call(
        _guidance_kernel,
        grid=(b // _BBLK,),
        in_specs=[
            pl.BlockSpec((protos.shape[0], c), lambda i: (0, 0)),
            pl.BlockSpec((_BBLK, c, hw), lambda i: (i, 0, 0)),
        ],
        out_specs=pl.BlockSpec((_BBLK, 1, hw), lambda i: (i, 0, 0)),
        out_shape=jax.ShapeDtypeStruct((b, 1, hw), jnp.float32),
        compiler_params=pltpu.CompilerParams(
            dimension_semantics=("parallel",),
        ),
    )(protos, x3)

    return out.reshape(b, 1, h, w)


# speedup vs baseline: 2.1774x; 1.0778x over previous
import jax
import jax.numpy as jnp
from jax.experimental import pallas as pl
from jax.experimental.pallas import tpu as pltpu

_EPS = 1e-12


_BBLK = 2


def _guidance_kernel(p_ref, x_ref, o_ref):
    for bi in range(_BBLK):
        xb = x_ref[bi]
        s = jnp.dot(p_ref[...], xb, preferred_element_type=jnp.float32)
        ev_f = jnp.max(s[:16], axis=0)
        ev_a = jnp.max(s[16:], axis=0)
        norm2 = jnp.sum(xb * xb, axis=0)
        norm = jnp.maximum(jnp.sqrt(norm2), _EPS)
        o_ref[bi] = ((ev_f - ev_a) / norm)[None, :]


def kernel(x, forgery_protos, authentic_protos):
    b, c, h, w = x.shape
    hw = h * w
    protos = jnp.concatenate([forgery_protos, authentic_protos], axis=0)
    x3 = x.reshape(b, c, hw)

    out = pl.pallas_call(
        _guidance_kernel,
        grid=(b // _BBLK,),
        in_specs=[
            pl.BlockSpec((protos.shape[0], c), lambda i: (0, 0)),
            pl.BlockSpec((_BBLK, c, hw), lambda i: (i, 0, 0)),
        ],
        out_specs=pl.BlockSpec((_BBLK, 1, hw), lambda i: (i, 0, 0)),
        out_shape=jax.ShapeDtypeStruct((b, 1, hw), jnp.float32),
        compiler_params=pltpu.CompilerParams(
            dimension_semantics=("parallel",),
        ),
    )(protos, x3)

    return out.reshape(b, 1, h, w)
